# Initial kernel scaffold; baseline (speedup 1.0000x reference)
import jax
import jax.numpy as jnp
from jax.experimental import pallas as pl


def kernel(m, edge_index, W_node, b_node, W_edge, b_edge):
    raise NotImplementedError("write your pallas kernel here")


# speedup vs baseline: 2.1763x; 2.1763x over previous
import functools

import jax
import jax.numpy as jnp
from jax import lax
from jax.experimental import pallas as pl
from jax.experimental.pallas import tpu as pltpu
from jax.experimental.pallas import tpu_sc as plsc

N_NODES = 10000
N_EDGES = 160000
FEATS = 256
HALF = 128

EDGE_CHUNK = 128
N_CHUNKS = N_EDGES // EDGE_CHUNK
NC = 2
NS = 16
ROW_CHUNK = 200
N_ROW_CHUNKS = N_NODES // ROW_CHUNK

_sc_mesh = plsc.VectorSubcoreMesh(core_axis_name="c", subcore_axis_name="s")


def _segment_sum_sc(m, dst, zeros_half):
    @functools.partial(
        pl.kernel,
        out_type=jax.ShapeDtypeStruct((N_NODES, FEATS), jnp.float32),
        mesh=_sc_mesh,
        scratch_types=[
            pltpu.VMEM((EDGE_CHUNK,), jnp.int32),
            pltpu.VMEM((EDGE_CHUNK, HALF), jnp.float32),
            pltpu.VMEM_SHARED((N_NODES, HALF), jnp.float32),
        ],
    )
    def k(m_hbm, dst_hbm, zeros_hbm, h_hbm, idx_v, rows_v, acc_sh):
        c = lax.axis_index("c")
        s = lax.axis_index("s")
        n_row_iter = (N_ROW_CHUNKS + NS - 1) // NS

        @pl.loop(0, n_row_iter)
        def _(i):
            rchunk = i * NS + s

            @pl.when(rchunk < N_ROW_CHUNKS)
            def _():
                r0 = rchunk * ROW_CHUNK
                pltpu.sync_copy(
                    zeros_hbm.at[pl.ds(r0, ROW_CHUNK)],
                    acc_sh.at[pl.ds(r0, ROW_CHUNK)],
                )

        plsc.subcore_barrier()

        n_iter = (N_CHUNKS + NS - 1) // NS

        @pl.loop(0, n_iter)
        def _(i):
            chunk = i * NS + s

            @pl.when(chunk < N_CHUNKS)
            def _():
                base = chunk * EDGE_CHUNK
                pltpu.sync_copy(dst_hbm.at[pl.ds(base, EDGE_CHUNK)], idx_v)
                pltpu.sync_copy(
                    m_hbm.at[pl.ds(base, EDGE_CHUNK), pl.ds(c * HALF, HALF)],
                    rows_v,
                )
                pltpu.sync_copy(rows_v, acc_sh.at[idx_v], add=True)

        plsc.subcore_barrier()

        @pl.loop(0, n_row_iter)
        def _(i):
            rchunk = i * NS + s

            @pl.when(rchunk < N_ROW_CHUNKS)
            def _():
                r0 = rchunk * ROW_CHUNK
                pltpu.sync_copy(
                    acc_sh.at[pl.ds(r0, ROW_CHUNK)],
                    h_hbm.at[pl.ds(r0, ROW_CHUNK), pl.ds(c * HALF, HALF)],
                )

    return k(m, dst, zeros_half)


def _gather_rows_sc(table, idx):
    @functools.partial(
        pl.kernel,
        out_type=jax.ShapeDtypeStruct((N_EDGES, FEATS), jnp.float32),
        mesh=_sc_mesh,
        scratch_types=[
            pltpu.VMEM((EDGE_CHUNK,), jnp.int32),
            pltpu.VMEM((EDGE_CHUNK, FEATS), jnp.float32),
            pltpu.SemaphoreType.DMA,
        ],
    )
    def k(tab_hbm, idx_hbm, g_hbm, idx_v, rows_v, sem):
        c = lax.axis_index("c")
        s = lax.axis_index("s")
        wid = s * NC + c
        nw = NC * NS
        n_iter = (N_CHUNKS + nw - 1) // nw

        @pl.loop(0, n_iter)
        def _(i):
            chunk = i * nw + wid

            @pl.when(chunk < N_CHUNKS)
            def _():
                base = chunk * EDGE_CHUNK
                pltpu.sync_copy(idx_hbm.at[pl.ds(base, EDGE_CHUNK)], idx_v)
                pltpu.async_copy(tab_hbm.at[idx_v], rows_v, sem).wait()
                pltpu.sync_copy(rows_v, g_hbm.at[pl.ds(base, EDGE_CHUNK)])

    return k(table, idx)


_NODE_BLK = 1000


def _node_tc(h, W_top, W_node, b_edge, b_node):
    def body(h_ref, wt_ref, wn_ref, be_ref, bn_ref, hwt_ref, hout_ref):
        hb = h_ref[...]
        hwt_ref[...] = (
            jnp.dot(hb, wt_ref[...], preferred_element_type=jnp.float32)
            + be_ref[...]
        )
        hout_ref[...] = jnp.maximum(
            jnp.dot(hb, wn_ref[...], preferred_element_type=jnp.float32)
            + bn_ref[...],
            0.0,
        )

    full = pl.BlockSpec((FEATS, FEATS), lambda i: (0, 0))
    bias = pl.BlockSpec((1, FEATS), lambda i: (0, 0))
    blk = pl.BlockSpec((_NODE_BLK, FEATS), lambda i: (i, 0))
    return pl.pallas_call(
        body,
        grid=(N_NODES // _NODE_BLK,),
        in_specs=[blk, full, full, bias, bias],
        out_specs=[blk, blk],
        out_shape=[
            jax.ShapeDtypeStruct((N_NODES, FEATS), jnp.float32),
            jax.ShapeDtypeStruct((N_NODES, FEATS), jnp.float32),
        ],
    )(h, W_top, W_node, b_edge, b_node)


_EDGE_BLK = 1000


def _edge_tc(m, W_bot, g):
    def body(m_ref, w_ref, g_ref, o_ref):
        o_ref[...] = jnp.maximum(
            jnp.dot(m_ref[...], w_ref[...], preferred_element_type=jnp.float32)
            + g_ref[...],
            0.0,
        )

    full = pl.BlockSpec((FEATS, FEATS), lambda i: (0, 0))
    blk = pl.BlockSpec((_EDGE_BLK, FEATS), lambda i: (i, 0))
    return pl.pallas_call(
        body,
        grid=(N_EDGES // _EDGE_BLK,),
        in_specs=[blk, full, blk],
        out_specs=blk,
        out_shape=jax.ShapeDtypeStruct((N_EDGES, FEATS), jnp.float32),
    )(m, W_bot, g)


def kernel(m, edge_index, W_node, b_node, W_edge, b_edge):
    src = edge_index[0].astype(jnp.int32)
    dst = edge_index[1].astype(jnp.int32)
    zeros_half = jnp.zeros((N_NODES, HALF), jnp.float32)

    h = _segment_sum_sc(m, dst, zeros_half)
    hWt, h_out = _node_tc(
        h,
        W_edge[:FEATS],
        W_node,
        b_edge.reshape(1, FEATS),
        b_node.reshape(1, FEATS),
    )
    g = _gather_rows_sc(hWt, src)
    m_out = _edge_tc(m, W_edge[FEATS:], g)
    return (m_out, h_out)


# speedup vs baseline: 2.8723x; 1.3198x over previous
import functools

import jax
import jax.numpy as jnp
from jax import lax
from jax.experimental import pallas as pl
from jax.experimental.pallas import tpu as pltpu
from jax.experimental.pallas import tpu_sc as plsc

N_NODES = 10000
N_EDGES = 160000
FEATS = 256
HALF = 128

EDGE_CHUNK = 128
N_CHUNKS = N_EDGES // EDGE_CHUNK
NC = 2
NS = 16
ROW_CHUNK = 200
N_ROW_CHUNKS = N_NODES // ROW_CHUNK

_sc_mesh = plsc.VectorSubcoreMesh(core_axis_name="c", subcore_axis_name="s")


def _segment_sum_sc(m, dst2, zeros_half):
    @functools.partial(
        pl.kernel,
        out_type=jax.ShapeDtypeStruct((N_NODES, FEATS), jnp.float32),
        mesh=_sc_mesh,
        scratch_types=[
            pltpu.VMEM_SHARED((N_NODES, HALF), jnp.float32),
        ],
    )
    def k(m_hbm, dst_hbm, zeros_hbm, h_hbm, acc_sh):
        c = lax.axis_index("c")
        s = lax.axis_index("s")
        n_row_iter = (N_ROW_CHUNKS + NS - 1) // NS

        @pl.loop(0, n_row_iter)
        def _(i):
            rchunk = i * NS + s

            @pl.when(rchunk < N_ROW_CHUNKS)
            def _():
                r0 = rchunk * ROW_CHUNK
                pltpu.sync_copy(
                    zeros_hbm.at[pl.ds(r0, ROW_CHUNK)],
                    acc_sh.at[pl.ds(r0, ROW_CHUNK)],
                )

        plsc.subcore_barrier()

        def body(idx_v, rows_v):
            pltpu.sync_copy(rows_v, acc_sh.at[idx_v.at[0]], add=True)

        def run_pipeline(col_blk):
            pltpu.emit_pipeline(
                body,
                grid=(N_CHUNKS,),
                in_specs=[
                    pl.BlockSpec((1, EDGE_CHUNK), lambda i: (0, i)),
                    pl.BlockSpec(
                        (EDGE_CHUNK, HALF), lambda i: (i, col_blk)
                    ),
                ],
                out_specs=[],
                core_axis_name="s",
                dimension_semantics=(pltpu.PARALLEL,),
            )(dst_hbm, m_hbm)

        @pl.when(c == 0)
        def _():
            run_pipeline(0)

        @pl.when(c == 1)
        def _():
            run_pipeline(1)

        plsc.subcore_barrier()

        @pl.loop(0, n_row_iter)
        def _(i):
            rchunk = i * NS + s

            @pl.when(rchunk < N_ROW_CHUNKS)
            def _():
                r0 = rchunk * ROW_CHUNK
                pltpu.sync_copy(
                    acc_sh.at[pl.ds(r0, ROW_CHUNK)],
                    h_hbm.at[pl.ds(r0, ROW_CHUNK), pl.ds(c * HALF, HALF)],
                )

    return k(m, dst2, zeros_half)


def _gather_rows_sc(table, idx2):
    @functools.partial(
        pl.kernel,
        out_type=jax.ShapeDtypeStruct((N_EDGES, FEATS), jnp.float32),
        mesh=_sc_mesh,
    )
    def k(tab_hbm, idx_hbm, g_hbm):
        def body(idx_v, rows_v):
            pltpu.sync_copy(tab_hbm.at[idx_v.at[0]], rows_v)

        pltpu.emit_pipeline(
            body,
            grid=(N_CHUNKS,),
            in_specs=[pl.BlockSpec((1, EDGE_CHUNK), lambda i: (0, i))],
            out_specs=[
                pl.BlockSpec((EDGE_CHUNK, FEATS), lambda i: (i, 0))
            ],
            core_axis_name=("c", "s"),
            dimension_semantics=(pltpu.PARALLEL,),
        )(idx_hbm, g_hbm)

    return k(table, idx2)


_NODE_BLK = 1000


def _node_tc(h, W_top, W_node, b_edge, b_node):
    def body(h_ref, wt_ref, wn_ref, be_ref, bn_ref, hwt_ref, hout_ref):
        hb = h_ref[...]
        hwt_ref[...] = (
            jnp.dot(hb, wt_ref[...], preferred_element_type=jnp.float32)
            + be_ref[...]
        )
        hout_ref[...] = jnp.maximum(
            jnp.dot(hb, wn_ref[...], preferred_element_type=jnp.float32)
            + bn_ref[...],
            0.0,
        )

    full = pl.BlockSpec((FEATS, FEATS), lambda i: (0, 0))
    bias = pl.BlockSpec((1, FEATS), lambda i: (0, 0))
    blk = pl.BlockSpec((_NODE_BLK, FEATS), lambda i: (i, 0))
    return pl.pallas_call(
        body,
        grid=(N_NODES // _NODE_BLK,),
        in_specs=[blk, full, full, bias, bias],
        out_specs=[blk, blk],
        out_shape=[
            jax.ShapeDtypeStruct((N_NODES, FEATS), jnp.float32),
            jax.ShapeDtypeStruct((N_NODES, FEATS), jnp.float32),
        ],
    )(h, W_top, W_node, b_edge, b_node)


_EDGE_BLK = 1000


def _edge_tc(m, W_bot, g):
    def body(m_ref, w_ref, g_ref, o_ref):
        o_ref[...] = jnp.maximum(
            jnp.dot(m_ref[...], w_ref[...], preferred_element_type=jnp.float32)
            + g_ref[...],
            0.0,
        )

    full = pl.BlockSpec((FEATS, FEATS), lambda i: (0, 0))
    blk = pl.BlockSpec((_EDGE_BLK, FEATS), lambda i: (i, 0))
    return pl.pallas_call(
        body,
        grid=(N_EDGES // _EDGE_BLK,),
        in_specs=[blk, full, blk],
        out_specs=blk,
        out_shape=jax.ShapeDtypeStruct((N_EDGES, FEATS), jnp.float32),
    )(m, W_bot, g)


def kernel(m, edge_index, W_node, b_node, W_edge, b_edge):
    src = edge_index[0].astype(jnp.int32).reshape(1, N_EDGES)
    dst = edge_index[1].astype(jnp.int32).reshape(1, N_EDGES)
    zeros_half = jnp.zeros((N_NODES, HALF), jnp.float32)

    h = _segment_sum_sc(m, dst, zeros_half)
    hWt, h_out = _node_tc(
        h,
        W_edge[:FEATS],
        W_node,
        b_edge.reshape(1, FEATS),
        b_node.reshape(1, FEATS),
    )
    g = _gather_rows_sc(hWt, src)

    m_out = _edge_tc(m, W_edge[FEATS:], g)
    return (m_out, h_out)
